# baseline (speedup 1.0000x reference)
import functools

import jax
import jax.numpy as jnp
from jax import lax
from jax.experimental import pallas as pl
from jax.experimental.pallas import tpu as pltpu
from jax.experimental.pallas import tpu_sc as plsc

_NUM_CORES = 2
_NUM_SUBCORES = 16
_NUM_WORKERS = _NUM_CORES * _NUM_SUBCORES

_NSEM = 16
_ROWS_PER_TILE = 63


@functools.partial(jax.jit, static_argnames=("n_rows", "d"))
def _gather_rows(table, idx_flat, n_rows, d):
    v = table.shape[0]
    n_per_t = n_rows // _NUM_WORKERS
    n_groups = n_per_t // _NSEM
    mesh = plsc.VectorSubcoreMesh(core_axis_name="c", subcore_axis_name="s")

    @functools.partial(
        pl.kernel,
        mesh=mesh,
        compiler_params=pltpu.CompilerParams(use_tc_tiling_on_sc=False),
        out_type=jax.ShapeDtypeStruct((n_rows, d), jnp.float32),
        scratch_types=[
            pltpu.VMEM((n_per_t,), jnp.int32),
            pltpu.VMEM_SHARED((16 * _ROWS_PER_TILE, d), jnp.float32),
            [pltpu.SemaphoreType.DMA for _ in range(_NSEM)],
        ],
    )
    def k(table_hbm, idx_hbm, out_hbm, idx_v, shared, sems):
        c = lax.axis_index("c")
        s = lax.axis_index("s")
        pos0 = (c * _NUM_SUBCORES + s) * n_per_t
        pltpu.sync_copy(idx_hbm.at[pl.ds(pos0, n_per_t)], idx_v)

        r0 = s * _ROWS_PER_TILE
        full = jnp.minimum(r0 + _ROWS_PER_TILE, v) - r0 == _ROWS_PER_TILE

        @pl.when(full)
        def _():
            pltpu.sync_copy(
                table_hbm.at[pl.ds(r0, _ROWS_PER_TILE)],
                shared.at[pl.ds(r0, _ROWS_PER_TILE)],
            )

        rem = v - (v // _ROWS_PER_TILE) * _ROWS_PER_TILE

        @pl.when(jnp.logical_not(full) & (r0 < v))
        def _():
            pltpu.sync_copy(
                table_hbm.at[pl.ds(v - rem, rem)],
                shared.at[pl.ds(v - rem, rem)],
            )

        plsc.subcore_barrier()

        def copy_row(r, i, sem):
            pltpu.async_copy(
                shared.at[pl.ds(r, 1)],
                out_hbm.at[pl.ds(pos0 + i, 1)],
                sem,
            )

        def wait_row(sem):
            pltpu.make_async_copy(
                shared.at[pl.ds(0, 1)],
                out_hbm.at[pl.ds(pos0, 1)],
                sem,
            ).wait()

        def body(g, carry):
            i0 = g * _NSEM
            v16 = idx_v[pl.ds(i0, _NSEM)]
            for b in range(_NSEM):
                @pl.when(g > 0)
                def _(b=b):
                    wait_row(sems[b])

                copy_row(v16[b], i0 + b, sems[b])
            return carry

        lax.fori_loop(0, n_groups, body, 0)
        for b in range(_NSEM):
            wait_row(sems[b])

    return k(table, idx_flat)


def kernel(table, idx):
    v, d = table.shape
    b, t = idx.shape
    out = _gather_rows(table, idx.reshape(b * t), b * t, d)
    return out.reshape(b, t, v)


# speedup vs baseline: 1.1066x; 1.1066x over previous
import functools

import jax
import jax.numpy as jnp
from jax import lax
from jax.experimental import pallas as pl
from jax.experimental.pallas import tpu as pltpu
from jax.experimental.pallas import tpu_sc as plsc

_NUM_CORES = 2
_NUM_SUBCORES = 16
_NUM_WORKERS = _NUM_CORES * _NUM_SUBCORES

_CHUNK = 16
_NBUF = 4
_NSEMD = 16
_DPG = 64
_ROWS_PER_TILE = 63


@functools.partial(jax.jit, static_argnames=("n_rows", "d"))
def _gather_rows(table, idx_flat, n_rows, d):
    v = table.shape[0]
    n_per_t = n_rows // _NUM_WORKERS
    n_groups = n_per_t // (_NBUF * _CHUNK + _DPG)
    n_stream = n_groups * _NBUF * _CHUNK
    n_direct_loop = n_groups * _DPG
    n_epilogue = n_per_t - n_stream - n_direct_loop
    assert n_epilogue % _NSEMD == 0
    mesh = plsc.VectorSubcoreMesh(core_axis_name="c", subcore_axis_name="s")

    @functools.partial(
        pl.kernel,
        mesh=mesh,
        compiler_params=pltpu.CompilerParams(use_tc_tiling_on_sc=False),
        out_type=jax.ShapeDtypeStruct((n_rows, d), jnp.float32),
        scratch_types=[
            pltpu.VMEM((n_per_t,), jnp.int32),
            pltpu.VMEM_SHARED((16 * _ROWS_PER_TILE, d), jnp.float32),
            [pltpu.VMEM((_CHUNK, d), jnp.float32) for _ in range(_NBUF)],
            [pltpu.SemaphoreType.DMA for _ in range(_NBUF)],
            [pltpu.SemaphoreType.DMA for _ in range(_NBUF)],
            [pltpu.SemaphoreType.DMA for _ in range(_NSEMD)],
        ],
    )
    def k(table_hbm, idx_hbm, out_hbm, idx_v, shared, bufs, gsems, ssems, dsems):
        c = lax.axis_index("c")
        s = lax.axis_index("s")
        pos0 = (c * _NUM_SUBCORES + s) * n_per_t
        pltpu.sync_copy(idx_hbm.at[pl.ds(pos0, n_per_t)], idx_v)

        r0 = s * _ROWS_PER_TILE
        full = jnp.minimum(r0 + _ROWS_PER_TILE, v) - r0 == _ROWS_PER_TILE

        @pl.when(full)
        def _():
            pltpu.sync_copy(
                table_hbm.at[pl.ds(r0, _ROWS_PER_TILE)],
                shared.at[pl.ds(r0, _ROWS_PER_TILE)],
            )

        rem = v - (v // _ROWS_PER_TILE) * _ROWS_PER_TILE

        @pl.when(jnp.logical_not(full) & (r0 < v))
        def _():
            pltpu.sync_copy(
                table_hbm.at[pl.ds(v - rem, rem)],
                shared.at[pl.ds(v - rem, rem)],
            )

        plsc.subcore_barrier()

        def gather(ch, buf, sem):
            pltpu.async_copy(
                shared.at[idx_v.at[pl.ds(ch * _CHUNK, _CHUNK)]], buf, sem
            )

        def scatter(buf, ch, sem):
            pltpu.async_copy(
                buf, out_hbm.at[pl.ds(pos0 + ch * _CHUNK, _CHUNK)], sem
            )

        def wait_gather(buf, sem):
            pltpu.make_async_copy(shared.at[pl.ds(0, _CHUNK)], buf, sem).wait()

        def wait_scatter(buf, sem):
            pltpu.make_async_copy(
                buf, out_hbm.at[pl.ds(pos0, _CHUNK)], sem
            ).wait()

        def copy_row(r, i, sem):
            pltpu.async_copy(
                shared.at[pl.ds(r, 1)], out_hbm.at[pl.ds(pos0 + i, 1)], sem
            )

        def wait_row(sem):
            pltpu.make_async_copy(
                shared.at[pl.ds(0, 1)], out_hbm.at[pl.ds(pos0, 1)], sem
            ).wait()

        for b in range(_NBUF):
            gather(b, bufs[b], gsems[b])

        def body(g, carry):
            c0 = g * _NBUF
            for b in range(_NBUF):
                wait_gather(bufs[b], gsems[b])
                scatter(bufs[b], c0 + b, ssems[b])

            d0 = n_stream + g * _DPG
            for kk in range(_DPG // _NSEMD):
                v16 = idx_v[pl.ds(d0 + kk * _NSEMD, _NSEMD)]
                for b in range(_NSEMD):
                    if kk > 0:
                        wait_row(dsems[b])
                    else:
                        @pl.when(g > 0)
                        def _(b=b):
                            wait_row(dsems[b])

                    copy_row(v16[b], d0 + kk * _NSEMD + b, dsems[b])

            for b in range(_NBUF):
                wait_scatter(bufs[b], ssems[b])

                @pl.when(g < n_groups - 1)
                def _(b=b):
                    gather(c0 + b + _NBUF, bufs[b], gsems[b])

            return carry

        lax.fori_loop(0, n_groups, body, 0)

        e0 = n_stream + n_direct_loop
        for kk in range(n_epilogue // _NSEMD):
            v16 = idx_v[pl.ds(e0 + kk * _NSEMD, _NSEMD)]
            for b in range(_NSEMD):
                wait_row(dsems[b])
                copy_row(v16[b], e0 + kk * _NSEMD + b, dsems[b])
        for b in range(_NSEMD):
            wait_row(dsems[b])

    return k(table, idx_flat)


def kernel(table, idx):
    v, d = table.shape
    b, t = idx.shape
    out = _gather_rows(table, idx.reshape(b * t), b * t, d)
    return out.reshape(b, t, v)


# speedup vs baseline: 1.1312x; 1.0222x over previous
import functools

import jax
import jax.numpy as jnp
from jax import lax
from jax.experimental import pallas as pl
from jax.experimental.pallas import tpu as pltpu
from jax.experimental.pallas import tpu_sc as plsc

_NUM_CORES = 2
_NUM_SUBCORES = 16
_NUM_WORKERS = _NUM_CORES * _NUM_SUBCORES

_CHUNK = 16
_NBUF = 4
_ROWS_PER_TILE = 63


@functools.partial(jax.jit, static_argnames=("n_rows", "d"))
def _gather_rows(table, idx_flat, n_rows, d):
    v = table.shape[0]
    n_per_t = n_rows // _NUM_WORKERS
    n_chunks = n_per_t // _CHUNK
    n_groups = n_chunks // _NBUF
    mesh = plsc.VectorSubcoreMesh(core_axis_name="c", subcore_axis_name="s")

    @functools.partial(
        pl.kernel,
        mesh=mesh,
        compiler_params=pltpu.CompilerParams(use_tc_tiling_on_sc=False),
        out_type=jax.ShapeDtypeStruct((n_rows, d), jnp.float32),
        scratch_types=[
            pltpu.VMEM((n_per_t,), jnp.int32),
            pltpu.VMEM_SHARED((16 * _ROWS_PER_TILE, d), jnp.float32),
            [pltpu.VMEM((_CHUNK, d), jnp.float32) for _ in range(_NBUF)],
            [pltpu.SemaphoreType.DMA for _ in range(_NBUF)],
            [pltpu.SemaphoreType.DMA for _ in range(_NBUF)],
        ],
    )
    def k(table_hbm, idx_hbm, out_hbm, idx_v, shared, bufs, gsems, ssems):
        c = lax.axis_index("c")
        s = lax.axis_index("s")
        pos0 = (c * _NUM_SUBCORES + s) * n_per_t
        pltpu.sync_copy(idx_hbm.at[pl.ds(pos0, n_per_t)], idx_v)

        for b in range(_NBUF):
            pltpu.async_copy(
                table_hbm.at[idx_v.at[pl.ds(b * _CHUNK, _CHUNK)]],
                bufs[b],
                gsems[b],
            )

        r0 = s * _ROWS_PER_TILE
        full = jnp.minimum(r0 + _ROWS_PER_TILE, v) - r0 == _ROWS_PER_TILE

        @pl.when(full)
        def _():
            pltpu.sync_copy(
                table_hbm.at[pl.ds(r0, _ROWS_PER_TILE)],
                shared.at[pl.ds(r0, _ROWS_PER_TILE)],
            )

        rem = v - (v // _ROWS_PER_TILE) * _ROWS_PER_TILE

        @pl.when(jnp.logical_not(full) & (r0 < v))
        def _():
            pltpu.sync_copy(
                table_hbm.at[pl.ds(v - rem, rem)],
                shared.at[pl.ds(v - rem, rem)],
            )

        def gather(ch, buf, sem):
            pltpu.async_copy(
                shared.at[idx_v.at[pl.ds(ch * _CHUNK, _CHUNK)]], buf, sem
            )

        def scatter(buf, ch, sem):
            pltpu.async_copy(
                buf,
                out_hbm.at[pl.ds(pos0 + ch * _CHUNK, _CHUNK)],
                sem,
            )

        def wait_gather(buf, sem):
            pltpu.make_async_copy(shared.at[pl.ds(0, _CHUNK)], buf, sem).wait()

        def wait_gather_hbm(buf, sem):
            pltpu.make_async_copy(
                table_hbm.at[pl.ds(0, _CHUNK)], buf, sem
            ).wait()

        def wait_scatter(buf, sem):
            pltpu.make_async_copy(
                buf, out_hbm.at[pl.ds(pos0, _CHUNK)], sem
            ).wait()

        for b in range(_NBUF):
            wait_gather_hbm(bufs[b], gsems[b])
            scatter(bufs[b], b, ssems[b])
        plsc.subcore_barrier()
        for b in range(_NBUF):
            wait_scatter(bufs[b], ssems[b])
            gather(b + _NBUF, bufs[b], gsems[b])

        def body(g, carry):
            c0 = g * _NBUF
            for b in range(_NBUF):
                wait_gather(bufs[b], gsems[b])
                scatter(bufs[b], c0 + b, ssems[b])
            for b in range(_NBUF):
                wait_scatter(bufs[b], ssems[b])

                @pl.when(g < n_groups - 1)
                def _(b=b):
                    gather(c0 + b + _NBUF, bufs[b], gsems[b])

            return carry

        lax.fori_loop(1, n_groups, body, 0)

    return k(table, idx_flat)


def kernel(table, idx):
    v, d = table.shape
    b, t = idx.shape
    out = _gather_rows(table, idx.reshape(b * t), b * t, d)
    return out.reshape(b, t, v)


# speedup vs baseline: 1.1315x; 1.0003x over previous
import functools

import jax
import jax.numpy as jnp
from jax import lax
from jax.experimental import pallas as pl
from jax.experimental.pallas import tpu as pltpu
from jax.experimental.pallas import tpu_sc as plsc

_NUM_CORES = 2
_NUM_SUBCORES = 16
_NUM_WORKERS = _NUM_CORES * _NUM_SUBCORES

_CHUNK = 16
_NBUF = 4
_ROWS_PER_TILE = 63


@functools.partial(jax.jit, static_argnames=("n_rows", "d"))
def _gather_rows(table, idx_flat, n_rows, d):
    v = table.shape[0]
    n_per_t = n_rows // _NUM_WORKERS
    n_chunks = n_per_t // _CHUNK
    n_groups = n_chunks // _NBUF
    mesh = plsc.VectorSubcoreMesh(core_axis_name="c", subcore_axis_name="s")

    @functools.partial(
        pl.kernel,
        mesh=mesh,
        compiler_params=pltpu.CompilerParams(use_tc_tiling_on_sc=False),
        out_type=jax.ShapeDtypeStruct((n_rows, d), jnp.float32),
        scratch_types=[
            pltpu.VMEM((n_per_t,), jnp.int32),
            pltpu.VMEM_SHARED((16 * _ROWS_PER_TILE, d), jnp.float32),
            [pltpu.VMEM((_CHUNK, d), jnp.float32) for _ in range(_NBUF)],
            [pltpu.SemaphoreType.DMA for _ in range(_NBUF)],
            [pltpu.SemaphoreType.DMA for _ in range(_NBUF)],
        ],
    )
    def k(table_hbm, idx_hbm, out_hbm, idx_v, shared, bufs, gsems, ssems):
        c = lax.axis_index("c")
        s = lax.axis_index("s")
        pos0 = (c * _NUM_SUBCORES + s) * n_per_t
        pltpu.sync_copy(idx_hbm.at[pl.ds(pos0, n_per_t)], idx_v)

        r0 = s * _ROWS_PER_TILE
        full = jnp.minimum(r0 + _ROWS_PER_TILE, v) - r0 == _ROWS_PER_TILE

        @pl.when(full)
        def _():
            pltpu.sync_copy(
                table_hbm.at[pl.ds(r0, _ROWS_PER_TILE)],
                shared.at[pl.ds(r0, _ROWS_PER_TILE)],
            )

        rem = v - (v // _ROWS_PER_TILE) * _ROWS_PER_TILE

        @pl.when(jnp.logical_not(full) & (r0 < v))
        def _():
            pltpu.sync_copy(
                table_hbm.at[pl.ds(v - rem, rem)],
                shared.at[pl.ds(v - rem, rem)],
            )

        plsc.subcore_barrier()

        def gather(ch, buf, sem):
            pltpu.async_copy(
                shared.at[idx_v.at[pl.ds(ch * _CHUNK, _CHUNK)]], buf, sem
            )

        def scatter(buf, ch, sem):
            pltpu.async_copy(
                buf,
                out_hbm.at[pl.ds(pos0 + ch * _CHUNK, _CHUNK)],
                sem,
            )

        def wait_gather(buf, sem):
            pltpu.make_async_copy(shared.at[pl.ds(0, _CHUNK)], buf, sem).wait()

        def wait_scatter(buf, sem):
            pltpu.make_async_copy(
                buf, out_hbm.at[pl.ds(pos0, _CHUNK)], sem
            ).wait()

        for b in range(_NBUF):
            gather(b, bufs[b], gsems[b])

        def body(g, carry):
            c0 = g * _NBUF
            for b in range(_NBUF):
                wait_gather(bufs[b], gsems[b])
                scatter(bufs[b], c0 + b, ssems[b])
            for b in range(_NBUF):
                wait_scatter(bufs[b], ssems[b])

                @pl.when(g < n_groups - 1)
                def _(b=b):
                    gather(c0 + b + _NBUF, bufs[b], gsems[b])

            return carry

        lax.fori_loop(0, n_groups, body, 0)

    return k(table, idx_flat)


def kernel(table, idx):
    v, d = table.shape
    b, t = idx.shape
    out = _gather_rows(table, idx.reshape(b * t), b * t, d)
    return out.reshape(b, t, v)
